# baseline (speedup 1.0000x reference)
import functools

import jax
import jax.numpy as jnp
from jax import lax
from jax.experimental import pallas as pl
from jax.experimental.pallas import tpu as pltpu
from jax.experimental.pallas import tpu_sc as plsc

NUM_FIELDS = 26
FIELD_SIZE = 100000
EMBED_DIM = 16
BATCH = 16384

NC, NS, L = 2, 16, 16
NW = NC * NS
N_ROWS = BATCH * NUM_FIELDS
ROWS_PER_W = N_ROWS // NW
CHUNK = 1664
N_CHUNKS = ROWS_PER_W // CHUNK

FEAT = NUM_FIELDS * EMBED_DIM
CHUNK_B = 128
B_PER_W = BATCH // NW
NB_CHUNKS = B_PER_W // CHUNK_B
NGRP = CHUNK_B // L


@functools.cache
def _make_sc_gather():
    mesh = plsc.VectorSubcoreMesh(
        core_axis_name="c", subcore_axis_name="s",
        num_cores=NC, num_subcores=NS)

    @functools.partial(
        pl.kernel,
        out_type=jax.ShapeDtypeStruct((BATCH, FEAT), jnp.float32),
        mesh=mesh,
        scratch_types=[
            pltpu.VMEM((CHUNK_B * NUM_FIELDS,), jnp.int32),
            pltpu.VMEM((NUM_FIELDS * CHUNK_B,), jnp.int32),
            pltpu.VMEM((NUM_FIELDS * CHUNK_B, EMBED_DIM), jnp.float32),
            pltpu.SemaphoreType.DMA,
        ],
        compiler_params=pltpu.CompilerParams(
            use_tc_tiling_on_sc=False, needs_layout_passes=False),
    )
    def _sc_gather(xs_hbm, table_hbm, out_hbm, idx_v, fidx_v, rows_v, sem):
        wid = lax.axis_index("s") * NC + lax.axis_index("c")
        b_base = wid * B_PER_W

        def body(ci, _):
            vlane = lax.broadcasted_iota(jnp.int32, (L,), 0)
            vbase = vlane * NUM_FIELDS
            b0 = b_base + ci * CHUNK_B
            pltpu.sync_copy(
                xs_hbm.at[pl.ds(b0 * NUM_FIELDS, CHUNK_B * NUM_FIELDS)],
                idx_v)
            for f in range(NUM_FIELDS):
                for g in range(NGRP):
                    src = vbase + (g * L * NUM_FIELDS + f)
                    vals = plsc.load_gather(idx_v, [src]) + f * FIELD_SIZE
                    fidx_v[pl.ds(f * CHUNK_B + g * L, L)] = vals
            copies = [
                pltpu.async_copy(
                    table_hbm.at[fidx_v.at[pl.ds(f * CHUNK_B, CHUNK_B)]],
                    rows_v.at[pl.ds(f * CHUNK_B, CHUNK_B)],
                    sem)
                for f in range(NUM_FIELDS)
            ]
            for c in copies:
                c.wait()
            for f in range(NUM_FIELDS):
                pltpu.sync_copy(
                    rows_v.at[pl.ds(f * CHUNK_B, CHUNK_B)],
                    out_hbm.at[pl.ds(b0, CHUNK_B),
                               pl.ds(f * EMBED_DIM, EMBED_DIM)])
            return 0

        lax.fori_loop(0, NB_CHUNKS, body, 0)

    return _sc_gather


TBL_ROWS = sum([FIELD_SIZE] * NUM_FIELDS)
RPK_BLK = 16384
RPK_GRID = -(-TBL_ROWS // RPK_BLK)


def _repack_body(xt_ref, o_ref):
    x = xt_ref[...]
    eye = jnp.equal(
        lax.broadcasted_iota(jnp.int32, (EMBED_DIM, EMBED_DIM), 0),
        lax.broadcasted_iota(jnp.int32, (EMBED_DIM, EMBED_DIM), 1),
    ).astype(jnp.float32)
    yt = lax.dot_general(x, eye, (((0,), (0,)), ((), ())),
                         preferred_element_type=jnp.float32)
    yt3 = yt.reshape(RPK_BLK // 8, 8, EMBED_DIM)
    for k in range(8):
        o_ref[:, pl.ds(k * EMBED_DIM, EMBED_DIM)] = yt3[:, k, :]


def _repack(table_t):
    return pl.pallas_call(
        _repack_body,
        grid=(RPK_GRID,),
        in_specs=[pl.BlockSpec((EMBED_DIM, RPK_BLK), lambda i: (0, i))],
        out_specs=pl.BlockSpec((RPK_BLK // 8, 128), lambda i: (i, 0)),
        out_shape=jax.ShapeDtypeStruct((TBL_ROWS * EMBED_DIM // 128, 128),
                                       jnp.float32),
    )(table_t)


BB = 1024


def _mlp_body(emb_ref, xd_ref, wlin_ref, w1_ref, b1_ref, w2_ref, b2_ref,
              w3_ref, b3_ref, w4_ref, bias_ref, out_ref):
    h = emb_ref[...]
    h = jnp.maximum(
        jnp.dot(h, w1_ref[...], preferred_element_type=jnp.float32)
        + b1_ref[...], 0.0)
    h = jnp.maximum(
        jnp.dot(h, w2_ref[...], preferred_element_type=jnp.float32)
        + b2_ref[...], 0.0)
    h = jnp.maximum(
        jnp.dot(h, w3_ref[...], preferred_element_type=jnp.float32)
        + b3_ref[...], 0.0)
    y_deep = jnp.sum(h * w4_ref[...], axis=1, keepdims=True)
    y_wide = jnp.sum(xd_ref[...] * wlin_ref[...], axis=1, keepdims=True)
    out_ref[...] = jax.nn.sigmoid(y_deep + y_wide + bias_ref[0, 0])


def _mlp(emb, x_dense, wlin_row, W1, b1, W2, b2, W3, b3, w4_row, bias):
    nb = BATCH // BB
    full = lambda shape: pl.BlockSpec(shape, lambda i: (0, 0))
    return pl.pallas_call(
        _mlp_body,
        grid=(nb,),
        in_specs=[
            pl.BlockSpec((BB, NUM_FIELDS * EMBED_DIM), lambda i: (i, 0)),
            pl.BlockSpec((BB, 13), lambda i: (i, 0)),
            full((1, 13)),
            full(W1.shape),
            full((1, 1024)),
            full(W2.shape),
            full((1, 512)),
            full(W3.shape),
            full((1, 256)),
            full((1, 256)),
            full((1, 1)),
        ],
        out_specs=pl.BlockSpec((BB, 1), lambda i: (i, 0)),
        out_shape=jax.ShapeDtypeStruct((BATCH, 1), jnp.float32),
    )(emb, x_dense, wlin_row, W1, b1, W2, b2, W3, b3, w4_row, bias)


def kernel(x_dense, x_sparse, W_lin, b_lin, table, W1, b1, W2, b2, W3, b3,
           W4, b4):
    xs_flat = x_sparse.astype(jnp.int32).reshape(-1)
    table_lin = _repack(table.T).reshape(TBL_ROWS, EMBED_DIM)
    emb = _make_sc_gather()(xs_flat, table_lin)
    bias = (b_lin + b4).reshape(1, 1)
    y = _mlp(emb, x_dense, W_lin.reshape(1, 13), W1, b1.reshape(1, 1024),
             W2, b2.reshape(1, 512), W3, b3.reshape(1, 256),
             W4.reshape(1, 256), bias)
    return y[:, 0]


# speedup vs baseline: 1.5019x; 1.5019x over previous
import functools

import jax
import jax.numpy as jnp
from jax import lax
from jax.experimental import pallas as pl
from jax.experimental.pallas import tpu as pltpu
from jax.experimental.pallas import tpu_sc as plsc

NUM_FIELDS = 26
FIELD_SIZE = 100000
EMBED_DIM = 16
BATCH = 16384

NC, NS, L = 2, 16, 16
NW = NC * NS
N_ROWS = BATCH * NUM_FIELDS
ROWS_PER_W = N_ROWS // NW
CHUNK = 1664
N_CHUNKS = ROWS_PER_W // CHUNK

FEAT = NUM_FIELDS * EMBED_DIM
CHUNK_B = 128
B_PER_W = BATCH // NW
NB_CHUNKS = B_PER_W // CHUNK_B
NGRP = CHUNK_B // L


@functools.cache
def _make_sc_gather():
    mesh = plsc.VectorSubcoreMesh(
        core_axis_name="c", subcore_axis_name="s",
        num_cores=NC, num_subcores=NS)

    @functools.partial(
        pl.kernel,
        out_type=jax.ShapeDtypeStruct((BATCH, FEAT), jnp.float32),
        mesh=mesh,
        scratch_types=[
            pltpu.VMEM((CHUNK_B * NUM_FIELDS,), jnp.int32),
            pltpu.VMEM((NUM_FIELDS * CHUNK_B,), jnp.int32),
            pltpu.VMEM((NUM_FIELDS * CHUNK_B, EMBED_DIM), jnp.float32),
            pltpu.SemaphoreType.DMA,
        ],
        compiler_params=pltpu.CompilerParams(
            use_tc_tiling_on_sc=False, needs_layout_passes=False),
    )
    def _sc_gather(xs_hbm, table_hbm, out_hbm, idx_v, fidx_v, rows_v, sem):
        wid = lax.axis_index("s") * NC + lax.axis_index("c")
        b_base = wid * B_PER_W

        def body(ci, _):
            vlane = lax.broadcasted_iota(jnp.int32, (L,), 0)
            vbase = vlane * NUM_FIELDS
            b0 = b_base + ci * CHUNK_B
            pltpu.sync_copy(
                xs_hbm.at[pl.ds(b0 * NUM_FIELDS, CHUNK_B * NUM_FIELDS)],
                idx_v)
            for f in range(NUM_FIELDS):
                for g in range(NGRP):
                    src = vbase + (g * L * NUM_FIELDS + f)
                    vals = plsc.load_gather(idx_v, [src]) + f * FIELD_SIZE
                    fidx_v[pl.ds(f * CHUNK_B + g * L, L)] = vals
            copies = [
                pltpu.async_copy(
                    table_hbm.at[fidx_v.at[pl.ds(f * CHUNK_B, CHUNK_B)]],
                    rows_v.at[pl.ds(f * CHUNK_B, CHUNK_B)],
                    sem)
                for f in range(NUM_FIELDS)
            ]
            for c in copies:
                c.wait()
            for f in range(NUM_FIELDS):
                pltpu.sync_copy(
                    rows_v.at[pl.ds(f * CHUNK_B, CHUNK_B)],
                    out_hbm.at[pl.ds(b0, CHUNK_B),
                               pl.ds(f * EMBED_DIM, EMBED_DIM)])
            return 0

        lax.fori_loop(0, NB_CHUNKS, body, 0)

    return _sc_gather


TBL_ROWS = sum([FIELD_SIZE] * NUM_FIELDS)
RPK_BLK = 16384
RPK_GRID = -(-TBL_ROWS // RPK_BLK)


def _repack_body(xt_ref, o_ref):
    x = xt_ref[...]
    emb = jnp.equal(
        lax.broadcasted_iota(jnp.int32, (EMBED_DIM, 128), 0),
        lax.broadcasted_iota(jnp.int32, (EMBED_DIM, 128), 1) % EMBED_DIM,
    ).astype(jnp.float32)
    yc = lax.dot_general(x, emb, (((0,), (0,)), ((), ())),
                         preferred_element_type=jnp.float32)
    yc3 = yc.reshape(RPK_BLK // 8, 8, 128)
    sel = jnp.equal(
        lax.broadcasted_iota(jnp.int32, (8, 128), 0),
        lax.broadcasted_iota(jnp.int32, (8, 128), 1) // EMBED_DIM,
    ).astype(jnp.float32)
    o_ref[...] = jnp.sum(yc3 * sel[None, :, :], axis=1)


def _repack(table_t):
    return pl.pallas_call(
        _repack_body,
        grid=(RPK_GRID,),
        in_specs=[pl.BlockSpec((EMBED_DIM, RPK_BLK), lambda i: (0, i))],
        out_specs=pl.BlockSpec((RPK_BLK // 8, 128), lambda i: (i, 0)),
        out_shape=jax.ShapeDtypeStruct((TBL_ROWS * EMBED_DIM // 128, 128),
                                       jnp.float32),
    )(table_t)


BB = 1024


def _mlp_body(emb_ref, xd_ref, wlin_ref, w1_ref, b1_ref, w2_ref, b2_ref,
              w3_ref, b3_ref, w4_ref, bias_ref, out_ref):
    h = emb_ref[...].astype(jnp.bfloat16)
    h = jnp.maximum(
        jnp.dot(h, w1_ref[...], preferred_element_type=jnp.float32)
        + b1_ref[...], 0.0).astype(jnp.bfloat16)
    h = jnp.maximum(
        jnp.dot(h, w2_ref[...], preferred_element_type=jnp.float32)
        + b2_ref[...], 0.0).astype(jnp.bfloat16)
    h = jnp.maximum(
        jnp.dot(h, w3_ref[...], preferred_element_type=jnp.float32)
        + b3_ref[...], 0.0)
    y_deep = jnp.sum(h * w4_ref[...], axis=1, keepdims=True)
    y_wide = jnp.sum(xd_ref[...] * wlin_ref[...], axis=1, keepdims=True)
    out_ref[...] = jax.nn.sigmoid(y_deep + y_wide + bias_ref[0, 0])


def _mlp(emb, x_dense, wlin_row, W1, b1, W2, b2, W3, b3, w4_row, bias):
    nb = BATCH // BB
    full = lambda shape: pl.BlockSpec(shape, lambda i: (0, 0))
    return pl.pallas_call(
        _mlp_body,
        grid=(nb,),
        in_specs=[
            pl.BlockSpec((BB, NUM_FIELDS * EMBED_DIM), lambda i: (i, 0)),
            pl.BlockSpec((BB, 13), lambda i: (i, 0)),
            full((1, 13)),
            full(W1.shape),
            full((1, 1024)),
            full(W2.shape),
            full((1, 512)),
            full(W3.shape),
            full((1, 256)),
            full((1, 256)),
            full((1, 1)),
        ],
        out_specs=pl.BlockSpec((BB, 1), lambda i: (i, 0)),
        out_shape=jax.ShapeDtypeStruct((BATCH, 1), jnp.float32),
    )(emb, x_dense, wlin_row, W1, b1, W2, b2, W3, b3, w4_row, bias)


def kernel(x_dense, x_sparse, W_lin, b_lin, table, W1, b1, W2, b2, W3, b3,
           W4, b4):
    xs_flat = x_sparse.astype(jnp.int32).reshape(-1)
    table_lin = _repack(table.T).reshape(TBL_ROWS, EMBED_DIM)
    emb = _make_sc_gather()(xs_flat, table_lin)
    bias = (b_lin + b4).reshape(1, 1)
    y = _mlp(emb, x_dense, W_lin.reshape(1, 13),
             W1.astype(jnp.bfloat16), b1.reshape(1, 1024),
             W2.astype(jnp.bfloat16), b2.reshape(1, 512),
             W3.astype(jnp.bfloat16), b3.reshape(1, 256),
             W4.reshape(1, 256), bias)
    return y[:, 0]


# speedup vs baseline: 1.5047x; 1.0019x over previous
import functools

import jax
import jax.numpy as jnp
from jax import lax
from jax.experimental import pallas as pl
from jax.experimental.pallas import tpu as pltpu
from jax.experimental.pallas import tpu_sc as plsc

NUM_FIELDS = 26
FIELD_SIZE = 100000
EMBED_DIM = 16
BATCH = 16384

NC, NS, L = 2, 16, 16
NW = NC * NS
N_ROWS = BATCH * NUM_FIELDS
ROWS_PER_W = N_ROWS // NW
CHUNK = 1664
N_CHUNKS = ROWS_PER_W // CHUNK

FEAT = NUM_FIELDS * EMBED_DIM
CHUNK_B = 128
B_PER_W = BATCH // NW
NB_CHUNKS = B_PER_W // CHUNK_B
NGRP = CHUNK_B // L


@functools.cache
def _make_sc_gather():
    mesh = plsc.VectorSubcoreMesh(
        core_axis_name="c", subcore_axis_name="s",
        num_cores=NC, num_subcores=NS)

    @functools.partial(
        pl.kernel,
        out_type=jax.ShapeDtypeStruct((BATCH, FEAT), jnp.float32),
        mesh=mesh,
        scratch_types=[
            pltpu.VMEM((CHUNK_B * NUM_FIELDS,), jnp.int32),
            pltpu.VMEM((NUM_FIELDS * CHUNK_B,), jnp.int32),
            pltpu.VMEM((NUM_FIELDS * CHUNK_B, EMBED_DIM), jnp.float32),
            pltpu.SemaphoreType.DMA,
        ],
        compiler_params=pltpu.CompilerParams(
            use_tc_tiling_on_sc=False, needs_layout_passes=False),
    )
    def _sc_gather(xs_hbm, table_hbm, out_hbm, idx_v, fidx_v, rows_v, sem):
        wid = lax.axis_index("s") * NC + lax.axis_index("c")
        b_base = wid * B_PER_W

        def body(ci, _):
            vlane = lax.broadcasted_iota(jnp.int32, (L,), 0)
            vbase = vlane * NUM_FIELDS
            b0 = b_base + ci * CHUNK_B
            pltpu.sync_copy(
                xs_hbm.at[pl.ds(b0 * NUM_FIELDS, CHUNK_B * NUM_FIELDS)],
                idx_v)
            for f in range(NUM_FIELDS):
                for g in range(NGRP):
                    src = vbase + (g * L * NUM_FIELDS + f)
                    vals = plsc.load_gather(idx_v, [src]) + f * FIELD_SIZE
                    fidx_v[pl.ds(f * CHUNK_B + g * L, L)] = vals
            copies = [
                pltpu.async_copy(
                    table_hbm.at[fidx_v.at[pl.ds(f * CHUNK_B, CHUNK_B)]],
                    rows_v.at[pl.ds(f * CHUNK_B, CHUNK_B)],
                    sem)
                for f in range(NUM_FIELDS)
            ]
            for c in copies:
                c.wait()
            for f in range(NUM_FIELDS):
                pltpu.sync_copy(
                    rows_v.at[pl.ds(f * CHUNK_B, CHUNK_B)],
                    out_hbm.at[pl.ds(b0, CHUNK_B),
                               pl.ds(f * EMBED_DIM, EMBED_DIM)])
            return 0

        lax.fori_loop(0, NB_CHUNKS, body, 0)

    return _sc_gather


TBL_ROWS = sum([FIELD_SIZE] * NUM_FIELDS)
RPK_BLK = 16384
RPK_GRID = -(-TBL_ROWS // RPK_BLK)


def _repack_body(xt_ref, o_ref):
    x = xt_ref[...]
    emb = jnp.equal(
        lax.broadcasted_iota(jnp.int32, (EMBED_DIM, 128), 0),
        lax.broadcasted_iota(jnp.int32, (EMBED_DIM, 128), 1) % EMBED_DIM,
    ).astype(jnp.float32)
    yc = lax.dot_general(x, emb, (((0,), (0,)), ((), ())),
                         preferred_element_type=jnp.float32)
    yc3 = yc.reshape(RPK_BLK // 8, 8, 128)
    sel = jnp.equal(
        lax.broadcasted_iota(jnp.int32, (8, 128), 0),
        lax.broadcasted_iota(jnp.int32, (8, 128), 1) // EMBED_DIM,
    ).astype(jnp.float32)
    o_ref[...] = jnp.sum(yc3 * sel[None, :, :], axis=1)


def _repack(table_t):
    return pl.pallas_call(
        _repack_body,
        grid=(RPK_GRID,),
        in_specs=[pl.BlockSpec((EMBED_DIM, RPK_BLK), lambda i: (0, i))],
        out_specs=pl.BlockSpec((RPK_BLK // 8, 128), lambda i: (i, 0)),
        out_shape=jax.ShapeDtypeStruct((TBL_ROWS * EMBED_DIM // 128, 128),
                                       jnp.float32),
    )(table_t)


BB = 1024


def _mlp_body(emb_ref, xd_ref, wlin_ref, w1_ref, b1_ref, w2_ref, b2_ref,
              w3_ref, b3_ref, w4_ref, bias_ref, out_ref):
    h = emb_ref[...]
    h = jnp.maximum(
        jnp.dot(h, w1_ref[...], preferred_element_type=jnp.float32)
        + b1_ref[...], 0.0)
    h = jnp.maximum(
        jnp.dot(h, w2_ref[...], preferred_element_type=jnp.float32)
        + b2_ref[...], 0.0)
    h = jnp.maximum(
        jnp.dot(h, w3_ref[...], preferred_element_type=jnp.float32)
        + b3_ref[...], 0.0)
    y_deep = jnp.sum(h * w4_ref[...], axis=1, keepdims=True)
    y_wide = jnp.sum(xd_ref[...] * wlin_ref[...], axis=1, keepdims=True)
    out_ref[...] = jax.nn.sigmoid(y_deep + y_wide + bias_ref[0, 0])


def _mlp(emb, x_dense, wlin_row, W1, b1, W2, b2, W3, b3, w4_row, bias):
    nb = BATCH // BB
    full = lambda shape: pl.BlockSpec(shape, lambda i: (0, 0))
    return pl.pallas_call(
        _mlp_body,
        grid=(nb,),
        in_specs=[
            pl.BlockSpec((BB, NUM_FIELDS * EMBED_DIM), lambda i: (i, 0)),
            pl.BlockSpec((BB, 13), lambda i: (i, 0)),
            full((1, 13)),
            full(W1.shape),
            full((1, 1024)),
            full(W2.shape),
            full((1, 512)),
            full(W3.shape),
            full((1, 256)),
            full((1, 256)),
            full((1, 1)),
        ],
        out_specs=pl.BlockSpec((BB, 1), lambda i: (i, 0)),
        out_shape=jax.ShapeDtypeStruct((BATCH, 1), jnp.float32),
    )(emb, x_dense, wlin_row, W1, b1, W2, b2, W3, b3, w4_row, bias)


def kernel(x_dense, x_sparse, W_lin, b_lin, table, W1, b1, W2, b2, W3, b3,
           W4, b4):
    xs_flat = x_sparse.astype(jnp.int32).reshape(-1)
    table_lin = _repack(table.T).reshape(TBL_ROWS, EMBED_DIM)
    emb = _make_sc_gather()(xs_flat, table_lin)
    bias = (b_lin + b4).reshape(1, 1)
    y = _mlp(emb, x_dense, W_lin.reshape(1, 13), W1, b1.reshape(1, 1024),
             W2, b2.reshape(1, 512), W3, b3.reshape(1, 256),
             W4.reshape(1, 256), bias)
    return y[:, 0]
